# Initial kernel scaffold; baseline (speedup 1.0000x reference)
import jax
import jax.numpy as jnp
from jax.experimental import pallas as pl


def kernel(x, edge_index, edge_attr, W1, b1, W2, b2, W3, b3):
    raise NotImplementedError("write your pallas kernel here")


# speedup vs baseline: 11.0061x; 11.0061x over previous
import functools

import jax
import jax.numpy as jnp
from jax import lax
from jax.experimental import pallas as pl
from jax.experimental.pallas import tpu as pltpu
from jax.experimental.pallas import tpu_sc as plsc

N = 10000
NP = 10240
E = 320000
D = 128

NC = 2
NS = 16
NW = NC * NS
EPW = E // NW
CH = 80
NCHUNK = EPW // CH
RPT = NP // NS

DPT = 640
NDEG = NP

_sc_mesh = plsc.VectorSubcoreMesh(core_axis_name="c", subcore_axis_name="s")


@functools.partial(
    pl.kernel,
    mesh=_sc_mesh,
    out_type=jax.ShapeDtypeStruct((NC * NDEG,), jnp.float32),
    scratch_types=[
        pltpu.VMEM((CH,), jnp.int32),
        pltpu.VMEM((DPT,), jnp.float32),
        pltpu.VMEM((CH,), jnp.float32),
        pltpu.VMEM_SHARED((NDEG,), jnp.float32),
    ],
)
def _sc_deg(dst_hbm, out_hbm, dstv, zbuf, ones, dacc):
    c = lax.axis_index("c")
    s = lax.axis_index("s")

    def fill(i, carry):
        zbuf[pl.ds(i * 16, 16)] = jnp.zeros((16,), jnp.float32)
        return carry
    lax.fori_loop(0, DPT // 16, fill, 0)

    def fill1(i, carry):
        ones[pl.ds(i * 16, 16)] = jnp.ones((16,), jnp.float32)
        return carry
    lax.fori_loop(0, CH // 16, fill1, 0)

    slab = pl.ds(s * DPT, DPT)
    pltpu.sync_copy(zbuf, dacc.at[slab])
    plsc.subcore_barrier()

    base = (c * NS + s) * EPW

    def body(i, carry):
        off = base + i * CH
        pltpu.sync_copy(dst_hbm.at[pl.ds(off, CH)], dstv)
        pltpu.sync_copy(ones, dacc.at[dstv], add=True)
        return carry
    lax.fori_loop(0, NCHUNK, body, 0)

    plsc.subcore_barrier()
    pltpu.sync_copy(dacc.at[slab], out_hbm.at[pl.ds(c * NDEG + s * DPT, DPT)])


@functools.partial(
    pl.kernel,
    mesh=_sc_mesh,
    out_type=jax.ShapeDtypeStruct((NC * NP, D), jnp.float32),
    scratch_types=[
        pltpu.VMEM((CH,), jnp.int32),
        pltpu.VMEM((CH,), jnp.int32),
        pltpu.VMEM((CH, D), jnp.float32),
        pltpu.VMEM_SHARED((NP, D), jnp.float32),
        pltpu.SemaphoreType.DMA,
    ],
)
def _sc_scatter(hs_hbm, src_hbm, dst_hbm, out_hbm, srcv, dstv, rows, acc, sem):
    c = lax.axis_index("c")
    s = lax.axis_index("s")

    slab = pl.ds(s * RPT, RPT)
    pltpu.sync_copy(hs_hbm.at[slab], acc.at[slab])
    plsc.subcore_barrier()

    base = (c * NS + s) * EPW

    def body(i, carry):
        off = base + i * CH
        pltpu.sync_copy(src_hbm.at[pl.ds(off, CH)], srcv)
        pltpu.async_copy(hs_hbm.at[srcv], rows, sem).wait()
        pltpu.sync_copy(dst_hbm.at[pl.ds(off, CH)], dstv)
        pltpu.sync_copy(rows, acc.at[dstv], add=True)
        return carry
    lax.fori_loop(0, NCHUNK, body, 0)

    plsc.subcore_barrier()
    pltpu.sync_copy(acc.at[slab], out_hbm.at[pl.ds(c * NP + s * RPT, RPT)])


def _tc_first(x, W1, d0, d1):
    def body(x_ref, w_ref, d0_ref, d1_ref, hs_ref, dinv_ref):
        deg = d0_ref[...] + d1_ref[...] + 1.0
        dinv = lax.rsqrt(deg)
        dinv_ref[...] = dinv
        h = jnp.dot(x_ref[...], w_ref[...],
                    preferred_element_type=jnp.float32,
                    precision=lax.Precision.HIGHEST)
        hs_ref[...] = h * dinv

    return pl.pallas_call(
        body,
        out_shape=(jax.ShapeDtypeStruct((NP, D), jnp.float32),
                   jax.ShapeDtypeStruct((NP, 1), jnp.float32)),
    )(x, W1, d0, d1)


def _tc_mid(a0, a1, hs, dinv, b, W):
    def body(a0_ref, a1_ref, hs_ref, dinv_ref, b_ref, w_ref, out_ref):
        dinv = dinv_ref[...]
        z = (a0_ref[...] + a1_ref[...] - hs_ref[...]) * dinv + b_ref[...]
        z = jnp.maximum(z, 0.0)
        out_ref[...] = jnp.dot(z, w_ref[...],
                               preferred_element_type=jnp.float32,
                               precision=lax.Precision.HIGHEST) * dinv

    return pl.pallas_call(
        body,
        out_shape=jax.ShapeDtypeStruct((NP, D), jnp.float32),
    )(a0, a1, hs, dinv, b, W)


def _tc_final(a0, a1, hs, dinv, b):
    def body(a0_ref, a1_ref, hs_ref, dinv_ref, b_ref, out_ref):
        out_ref[...] = ((a0_ref[...] + a1_ref[...] - hs_ref[...])
                        * dinv_ref[...] + b_ref[...])

    return pl.pallas_call(
        body,
        out_shape=jax.ShapeDtypeStruct((NP, D), jnp.float32),
    )(a0, a1, hs, dinv, b)


def kernel(x, edge_index, edge_attr, W1, b1, W2, b2, W3, b3):
    src = edge_index[0]
    dst = edge_index[1]
    b1r = b1.reshape(1, D)
    b2r = b2.reshape(1, D)
    b3r = b3.reshape(1, D)

    x_pad = jnp.pad(x, ((0, NP - N), (0, 0)))

    deg = _sc_deg(dst)
    d0 = deg[:NP].reshape(NP, 1)
    d1 = deg[NP:].reshape(NP, 1)

    hs1, dinv = _tc_first(x_pad, W1, d0, d1)
    acc = _sc_scatter(hs1, src, dst)
    hs2 = _tc_mid(acc[:NP], acc[NP:], hs1, dinv, b1r, W2)
    acc = _sc_scatter(hs2, src, dst)
    hs3 = _tc_mid(acc[:NP], acc[NP:], hs2, dinv, b2r, W3)
    acc = _sc_scatter(hs3, src, dst)
    out = _tc_final(acc[:NP], acc[NP:], hs3, dinv, b3r)
    return out[:N]


# speedup vs baseline: 18.9556x; 1.7223x over previous
import functools

import jax
import jax.numpy as jnp
from jax import lax
from jax.experimental import pallas as pl
from jax.experimental.pallas import tpu as pltpu
from jax.experimental.pallas import tpu_sc as plsc

N = 10000
NP = 10240
E = 320000
D = 128

NC = 2
NS = 16
NW = NC * NS
EPW = E // NW
CH = 80
NCHUNK = EPW // CH
RPT = NP // NS

DPT = 640
NDEG = NP

_sc_mesh = plsc.VectorSubcoreMesh(core_axis_name="c", subcore_axis_name="s")


@functools.partial(
    pl.kernel,
    mesh=_sc_mesh,
    out_type=jax.ShapeDtypeStruct((NC * NDEG,), jnp.float32),
    scratch_types=[
        pltpu.VMEM((CH,), jnp.int32),
        pltpu.VMEM((DPT,), jnp.float32),
        pltpu.VMEM((CH,), jnp.float32),
        pltpu.VMEM_SHARED((NDEG,), jnp.float32),
    ],
)
def _sc_deg(dst_hbm, out_hbm, dstv, zbuf, ones, dacc):
    c = lax.axis_index("c")
    s = lax.axis_index("s")

    def fill(i, carry):
        zbuf[pl.ds(i * 16, 16)] = jnp.zeros((16,), jnp.float32)
        return carry
    lax.fori_loop(0, DPT // 16, fill, 0)

    def fill1(i, carry):
        ones[pl.ds(i * 16, 16)] = jnp.ones((16,), jnp.float32)
        return carry
    lax.fori_loop(0, CH // 16, fill1, 0)

    slab = pl.ds(s * DPT, DPT)
    pltpu.sync_copy(zbuf, dacc.at[slab])
    plsc.subcore_barrier()

    base = (c * NS + s) * EPW

    def body(i, carry):
        off = base + i * CH
        pltpu.sync_copy(dst_hbm.at[pl.ds(off, CH)], dstv)
        pltpu.sync_copy(ones, dacc.at[dstv], add=True)
        return carry
    lax.fori_loop(0, NCHUNK, body, 0)

    plsc.subcore_barrier()
    pltpu.sync_copy(dacc.at[slab], out_hbm.at[pl.ds(c * NDEG + s * DPT, DPT)])


@functools.partial(
    pl.kernel,
    mesh=_sc_mesh,
    out_type=jax.ShapeDtypeStruct((NC * NP, D), jnp.float32),
    scratch_types=[
        pltpu.VMEM((EPW,), jnp.int32),
        pltpu.VMEM((2, CH), jnp.int32),
        pltpu.VMEM((2, CH, D), jnp.float32),
        pltpu.VMEM_SHARED((NP, D), jnp.float32),
        pltpu.SemaphoreType.DMA,
    ],
)
def _sc_scatter(hs_hbm, src_hbm, dst_hbm, out_hbm, srcall, dstv, rows, acc, sem):
    c = lax.axis_index("c")
    s = lax.axis_index("s")
    base = (c * NS + s) * EPW

    slab = pl.ds(s * RPT, RPT)
    init = pltpu.async_copy(hs_hbm.at[slab], acc.at[slab], sem)
    pltpu.sync_copy(src_hbm.at[pl.ds(base, EPW)], srcall)
    init.wait()
    plsc.subcore_barrier()

    def body(j, carry):
        p = lax.rem(j, 2)
        q = 1 - p
        gh = pltpu.async_copy(
            hs_hbm.at[srcall.at[pl.ds(j * CH, CH)]], rows.at[p], sem)
        pltpu.sync_copy(dst_hbm.at[pl.ds(base + j * CH, CH)], dstv.at[p])

        @pl.when(j > 0)
        def _():
            pltpu.sync_copy(rows.at[q], acc.at[dstv.at[q]], add=True)
        gh.wait()
        return carry
    lax.fori_loop(0, NCHUNK, body, 0)
    last = (NCHUNK - 1) % 2
    pltpu.sync_copy(rows.at[last], acc.at[dstv.at[last]], add=True)

    plsc.subcore_barrier()
    pltpu.sync_copy(acc.at[slab], out_hbm.at[pl.ds(c * NP + s * RPT, RPT)])


def _tc_first(x, W1, d0, d1):
    def body(x_ref, w_ref, d0_ref, d1_ref, hs_ref, dinv_ref):
        deg = d0_ref[...] + d1_ref[...] + 1.0
        dinv = lax.rsqrt(deg)
        dinv_ref[...] = dinv
        h = jnp.dot(x_ref[...], w_ref[...],
                    preferred_element_type=jnp.float32,
                    precision=lax.Precision.HIGHEST)
        hs_ref[...] = h * dinv

    return pl.pallas_call(
        body,
        out_shape=(jax.ShapeDtypeStruct((NP, D), jnp.float32),
                   jax.ShapeDtypeStruct((NP, 1), jnp.float32)),
    )(x, W1, d0, d1)


def _tc_mid(a0, a1, hs, dinv, b, W):
    def body(a0_ref, a1_ref, hs_ref, dinv_ref, b_ref, w_ref, out_ref):
        dinv = dinv_ref[...]
        z = (a0_ref[...] + a1_ref[...] - hs_ref[...]) * dinv + b_ref[...]
        z = jnp.maximum(z, 0.0)
        out_ref[...] = jnp.dot(z, w_ref[...],
                               preferred_element_type=jnp.float32,
                               precision=lax.Precision.HIGHEST) * dinv

    return pl.pallas_call(
        body,
        out_shape=jax.ShapeDtypeStruct((NP, D), jnp.float32),
    )(a0, a1, hs, dinv, b, W)


def _tc_final(a0, a1, hs, dinv, b):
    def body(a0_ref, a1_ref, hs_ref, dinv_ref, b_ref, out_ref):
        out_ref[...] = ((a0_ref[...] + a1_ref[...] - hs_ref[...])
                        * dinv_ref[...] + b_ref[...])

    return pl.pallas_call(
        body,
        out_shape=jax.ShapeDtypeStruct((NP, D), jnp.float32),
    )(a0, a1, hs, dinv, b)


def kernel(x, edge_index, edge_attr, W1, b1, W2, b2, W3, b3):
    src = edge_index[0]
    dst = edge_index[1]
    b1r = b1.reshape(1, D)
    b2r = b2.reshape(1, D)
    b3r = b3.reshape(1, D)

    x_pad = jnp.pad(x, ((0, NP - N), (0, 0)))

    deg = _sc_deg(dst)
    d0 = deg[:NP].reshape(NP, 1)
    d1 = deg[NP:].reshape(NP, 1)

    hs1, dinv = _tc_first(x_pad, W1, d0, d1)
    acc = _sc_scatter(hs1, src, dst)
    hs2 = _tc_mid(acc[:NP], acc[NP:], hs1, dinv, b1r, W2)
    acc = _sc_scatter(hs2, src, dst)
    hs3 = _tc_mid(acc[:NP], acc[NP:], hs2, dinv, b2r, W3)
    acc = _sc_scatter(hs3, src, dst)
    out = _tc_final(acc[:NP], acc[NP:], hs3, dinv, b3r)
    return out[:N]


# speedup vs baseline: 19.2235x; 1.0141x over previous
import functools

import jax
import jax.numpy as jnp
from jax import lax
from jax.experimental import pallas as pl
from jax.experimental.pallas import tpu as pltpu
from jax.experimental.pallas import tpu_sc as plsc

N = 10000
NP = 10240
E = 320000
D = 128

NC = 2
NS = 16
NW = NC * NS
EPW = E // NW
CH = 80
NCHUNK = EPW // CH
RPT = NP // NS

DPT = 640
NDEG = NP

_sc_mesh = plsc.VectorSubcoreMesh(core_axis_name="c", subcore_axis_name="s")


@functools.partial(
    pl.kernel,
    mesh=_sc_mesh,
    out_type=jax.ShapeDtypeStruct((NC * NDEG,), jnp.float32),
    scratch_types=[
        pltpu.VMEM((2, CH), jnp.int32),
        pltpu.VMEM((DPT,), jnp.float32),
        pltpu.VMEM((CH,), jnp.float32),
        pltpu.VMEM_SHARED((NDEG,), jnp.float32),
        pltpu.SemaphoreType.DMA,
    ],
)
def _sc_deg(dst_hbm, out_hbm, dstv, zbuf, ones, dacc, sems):
    c = lax.axis_index("c")
    s = lax.axis_index("s")

    def fill(i, carry):
        zbuf[pl.ds(i * 16, 16)] = jnp.zeros((16,), jnp.float32)
        return carry
    lax.fori_loop(0, DPT // 16, fill, 0)

    def fill1(i, carry):
        ones[pl.ds(i * 16, 16)] = jnp.ones((16,), jnp.float32)
        return carry
    lax.fori_loop(0, CH // 16, fill1, 0)

    slab = pl.ds(s * DPT, DPT)
    pltpu.sync_copy(zbuf, dacc.at[slab])
    plsc.subcore_barrier()

    base = (c * NS + s) * EPW

    def _scatter_wait():
        pltpu.make_async_copy(ones, dacc.at[dstv.at[0]], sems).wait()

    pltpu.sync_copy(dst_hbm.at[pl.ds(base, CH)], dstv.at[0])

    def body(j, carry):
        p = lax.rem(j, 2)
        q = 1 - p
        pltpu.async_copy(ones, dacc.at[dstv.at[q]], sems, add=True)

        @pl.when(j >= 2)
        def _():
            _scatter_wait()
        pltpu.sync_copy(dst_hbm.at[pl.ds(base + j * CH, CH)], dstv.at[p])
        return carry
    lax.fori_loop(1, NCHUNK, body, 0)
    last = (NCHUNK - 1) % 2
    pltpu.async_copy(ones, dacc.at[dstv.at[last]], sems, add=True)
    _scatter_wait()
    _scatter_wait()

    plsc.subcore_barrier()
    pltpu.sync_copy(dacc.at[slab], out_hbm.at[pl.ds(c * NDEG + s * DPT, DPT)])


@functools.partial(
    pl.kernel,
    mesh=_sc_mesh,
    out_type=jax.ShapeDtypeStruct((NC * NP, D), jnp.float32),
    scratch_types=[
        pltpu.VMEM((EPW,), jnp.int32),
        pltpu.VMEM((2, CH), jnp.int32),
        pltpu.VMEM((2, CH, D), jnp.float32),
        pltpu.VMEM_SHARED((NP, D), jnp.float32),
        pltpu.SemaphoreType.DMA,
        pltpu.SemaphoreType.DMA,
    ],
)
def _sc_scatter(hs_hbm, src_hbm, dst_hbm, out_hbm, srcall, dstv, rows, acc,
                semg, sems):
    c = lax.axis_index("c")
    s = lax.axis_index("s")
    base = (c * NS + s) * EPW

    slab = pl.ds(s * RPT, RPT)
    init = pltpu.async_copy(hs_hbm.at[slab], acc.at[slab], semg)
    pltpu.sync_copy(src_hbm.at[pl.ds(base, EPW)], srcall)
    init.wait()
    plsc.subcore_barrier()

    def _scatter_wait():
        pltpu.make_async_copy(rows.at[0], acc.at[dstv.at[0]], sems).wait()

    pltpu.async_copy(hs_hbm.at[srcall.at[pl.ds(0, CH)]], rows.at[0], semg).wait()
    pltpu.sync_copy(dst_hbm.at[pl.ds(base, CH)], dstv.at[0])

    def body(j, carry):
        p = lax.rem(j, 2)
        q = 1 - p
        pltpu.async_copy(rows.at[q], acc.at[dstv.at[q]], sems, add=True)

        @pl.when(j >= 2)
        def _():
            _scatter_wait()
        gh = pltpu.async_copy(
            hs_hbm.at[srcall.at[pl.ds(j * CH, CH)]], rows.at[p], semg)
        pltpu.sync_copy(dst_hbm.at[pl.ds(base + j * CH, CH)], dstv.at[p])
        gh.wait()
        return carry
    lax.fori_loop(1, NCHUNK, body, 0)
    last = (NCHUNK - 1) % 2
    pltpu.async_copy(rows.at[last], acc.at[dstv.at[last]], sems, add=True)
    _scatter_wait()
    _scatter_wait()

    plsc.subcore_barrier()
    pltpu.sync_copy(acc.at[slab], out_hbm.at[pl.ds(c * NP + s * RPT, RPT)])


def _tc_first(x, W1, d0, d1):
    def body(x_ref, w_ref, d0_ref, d1_ref, hs_ref, dinv_ref):
        deg = d0_ref[...] + d1_ref[...] + 1.0
        dinv = lax.rsqrt(deg)
        dinv_ref[...] = dinv
        h = jnp.dot(x_ref[...], w_ref[...],
                    preferred_element_type=jnp.float32,
                    precision=lax.Precision.HIGHEST)
        hs_ref[...] = h * dinv

    return pl.pallas_call(
        body,
        out_shape=(jax.ShapeDtypeStruct((NP, D), jnp.float32),
                   jax.ShapeDtypeStruct((NP, 1), jnp.float32)),
    )(x, W1, d0, d1)


def _tc_mid(a0, a1, hs, dinv, b, W):
    def body(a0_ref, a1_ref, hs_ref, dinv_ref, b_ref, w_ref, out_ref):
        dinv = dinv_ref[...]
        z = (a0_ref[...] + a1_ref[...] - hs_ref[...]) * dinv + b_ref[...]
        z = jnp.maximum(z, 0.0)
        out_ref[...] = jnp.dot(z, w_ref[...],
                               preferred_element_type=jnp.float32,
                               precision=lax.Precision.HIGHEST) * dinv

    return pl.pallas_call(
        body,
        out_shape=jax.ShapeDtypeStruct((NP, D), jnp.float32),
    )(a0, a1, hs, dinv, b, W)


def _tc_final(a0, a1, hs, dinv, b):
    def body(a0_ref, a1_ref, hs_ref, dinv_ref, b_ref, out_ref):
        out_ref[...] = ((a0_ref[...] + a1_ref[...] - hs_ref[...])
                        * dinv_ref[...] + b_ref[...])

    return pl.pallas_call(
        body,
        out_shape=jax.ShapeDtypeStruct((NP, D), jnp.float32),
    )(a0, a1, hs, dinv, b)


def kernel(x, edge_index, edge_attr, W1, b1, W2, b2, W3, b3):
    src = edge_index[0]
    dst = edge_index[1]
    b1r = b1.reshape(1, D)
    b2r = b2.reshape(1, D)
    b3r = b3.reshape(1, D)

    x_pad = jnp.pad(x, ((0, NP - N), (0, 0)))

    deg = _sc_deg(dst)
    d0 = deg[:NP].reshape(NP, 1)
    d1 = deg[NP:].reshape(NP, 1)

    hs1, dinv = _tc_first(x_pad, W1, d0, d1)
    acc = _sc_scatter(hs1, src, dst)
    hs2 = _tc_mid(acc[:NP], acc[NP:], hs1, dinv, b1r, W2)
    acc = _sc_scatter(hs2, src, dst)
    hs3 = _tc_mid(acc[:NP], acc[NP:], hs2, dinv, b2r, W3)
    acc = _sc_scatter(hs3, src, dst)
    out = _tc_final(acc[:NP], acc[NP:], hs3, dinv, b3r)
    return out[:N]


# speedup vs baseline: 25.5660x; 1.3299x over previous
import functools

import jax
import jax.numpy as jnp
from jax import lax
from jax.experimental import pallas as pl
from jax.experimental.pallas import tpu as pltpu
from jax.experimental.pallas import tpu_sc as plsc

N = 10000
NP = 10240
E = 320000
D = 128

NC = 2
NS = 16
NW = NC * NS
EPW = E // NW
CH = 80
NCHUNK = EPW // CH
RPT = NP // NS

DPT = 640
NDEG = NP

_sc_mesh = plsc.VectorSubcoreMesh(core_axis_name="c", subcore_axis_name="s")


@functools.partial(
    pl.kernel,
    mesh=_sc_mesh,
    out_type=jax.ShapeDtypeStruct((NC * NDEG,), jnp.float32),
    scratch_types=[
        pltpu.VMEM((2, CH), jnp.int32),
        pltpu.VMEM((DPT,), jnp.float32),
        pltpu.VMEM((CH,), jnp.float32),
        pltpu.VMEM_SHARED((NDEG,), jnp.float32),
        pltpu.SemaphoreType.DMA,
    ],
)
def _sc_deg(dst_hbm, out_hbm, dstv, zbuf, ones, dacc, sems):
    c = lax.axis_index("c")
    s = lax.axis_index("s")

    def fill(i, carry):
        zbuf[pl.ds(i * 16, 16)] = jnp.zeros((16,), jnp.float32)
        return carry
    lax.fori_loop(0, DPT // 16, fill, 0)

    def fill1(i, carry):
        ones[pl.ds(i * 16, 16)] = jnp.ones((16,), jnp.float32)
        return carry
    lax.fori_loop(0, CH // 16, fill1, 0)

    slab = pl.ds(s * DPT, DPT)
    pltpu.sync_copy(zbuf, dacc.at[slab])
    plsc.subcore_barrier()

    base = (c * NS + s) * EPW

    def _scatter_wait():
        pltpu.make_async_copy(ones, dacc.at[dstv.at[0]], sems).wait()

    pltpu.sync_copy(dst_hbm.at[pl.ds(base, CH)], dstv.at[0])

    def body(j, carry):
        p = lax.rem(j, 2)
        q = 1 - p
        pltpu.async_copy(ones, dacc.at[dstv.at[q]], sems, add=True)

        @pl.when(j >= 2)
        def _():
            _scatter_wait()
        pltpu.sync_copy(dst_hbm.at[pl.ds(base + j * CH, CH)], dstv.at[p])
        return carry
    lax.fori_loop(1, NCHUNK, body, 0)
    last = (NCHUNK - 1) % 2
    pltpu.async_copy(ones, dacc.at[dstv.at[last]], sems, add=True)
    _scatter_wait()
    _scatter_wait()

    plsc.subcore_barrier()
    pltpu.sync_copy(dacc.at[slab], out_hbm.at[pl.ds(c * NDEG + s * DPT, DPT)])


@functools.partial(
    pl.kernel,
    mesh=_sc_mesh,
    out_type=jax.ShapeDtypeStruct((NC * NP, D), jnp.float32),
    scratch_types=[
        pltpu.VMEM((8, CH), jnp.int32),
        pltpu.VMEM((8, CH), jnp.int32),
        pltpu.VMEM((4, CH, D), jnp.float32),
        pltpu.VMEM_SHARED((NP, D), jnp.float32),
        pltpu.SemaphoreType.DMA,
        pltpu.SemaphoreType.DMA,
        pltpu.SemaphoreType.DMA,
    ],
)
def _sc_scatter(hs_hbm, src_hbm, dst_hbm, out_hbm, srcv, dstv, rows, acc,
                semg, semi, sems):
    c = lax.axis_index("c")
    s = lax.axis_index("s")
    base = (c * NS + s) * EPW

    slab = pl.ds(s * RPT, RPT)
    pltpu.sync_copy(hs_hbm.at[slab], acc.at[slab])
    plsc.subcore_barrier()

    def _issue_idx(j, r):
        pltpu.async_copy(src_hbm.at[pl.ds(base + j * CH, CH)],
                         srcv.at[r], semi)
        pltpu.async_copy(dst_hbm.at[pl.ds(base + j * CH, CH)],
                         dstv.at[r], semi)

    def _idx_wait():
        pltpu.make_async_copy(src_hbm.at[pl.ds(base, CH)],
                              srcv.at[0], semi).wait()
        pltpu.make_async_copy(dst_hbm.at[pl.ds(base, CH)],
                              dstv.at[0], semi).wait()

    def _gather_wait():
        pltpu.make_async_copy(hs_hbm.at[srcv.at[0]], rows.at[0], semg).wait()

    def _scatter_wait():
        pltpu.make_async_copy(rows.at[0], acc.at[dstv.at[0]], sems).wait()

    _issue_idx(0, 0)
    _issue_idx(1, 1)
    _issue_idx(2, 2)
    _issue_idx(3, 3)
    _idx_wait()
    pltpu.async_copy(hs_hbm.at[srcv.at[0]], rows.at[0], semg)
    _idx_wait()
    pltpu.async_copy(hs_hbm.at[srcv.at[1]], rows.at[1], semg)

    def body(j, carry):
        p = lax.rem(j, 4)
        r = lax.rem(j, 8)
        _gather_wait()
        pltpu.async_copy(rows.at[p], acc.at[dstv.at[r]], sems, add=True)

        @pl.when(j >= 2)
        def _():
            _scatter_wait()

        @pl.when(j < NCHUNK - 2)
        def _():
            _idx_wait()
            pltpu.async_copy(hs_hbm.at[srcv.at[lax.rem(j + 2, 8)]],
                             rows.at[lax.rem(j + 2, 4)], semg)

        @pl.when(j < NCHUNK - 4)
        def _():
            _issue_idx(j + 4, lax.rem(j + 4, 8))
        return carry
    lax.fori_loop(0, NCHUNK, body, 0)
    _scatter_wait()
    _scatter_wait()

    plsc.subcore_barrier()
    pltpu.sync_copy(acc.at[slab], out_hbm.at[pl.ds(c * NP + s * RPT, RPT)])


def _tc_first(x, W1, d0, d1):
    def body(x_ref, w_ref, d0_ref, d1_ref, hs_ref, dinv_ref):
        deg = d0_ref[...] + d1_ref[...] + 1.0
        dinv = lax.rsqrt(deg)
        dinv_ref[...] = dinv
        h = jnp.dot(x_ref[...], w_ref[...],
                    preferred_element_type=jnp.float32,
                    precision=lax.Precision.HIGHEST)
        hs_ref[...] = h * dinv

    return pl.pallas_call(
        body,
        out_shape=(jax.ShapeDtypeStruct((NP, D), jnp.float32),
                   jax.ShapeDtypeStruct((NP, 1), jnp.float32)),
    )(x, W1, d0, d1)


def _tc_mid(a0, a1, hs, dinv, b, W):
    def body(a0_ref, a1_ref, hs_ref, dinv_ref, b_ref, w_ref, out_ref):
        dinv = dinv_ref[...]
        z = (a0_ref[...] + a1_ref[...] - hs_ref[...]) * dinv + b_ref[...]
        z = jnp.maximum(z, 0.0)
        out_ref[...] = jnp.dot(z, w_ref[...],
                               preferred_element_type=jnp.float32,
                               precision=lax.Precision.HIGHEST) * dinv

    return pl.pallas_call(
        body,
        out_shape=jax.ShapeDtypeStruct((NP, D), jnp.float32),
    )(a0, a1, hs, dinv, b, W)


def _tc_final(a0, a1, hs, dinv, b):
    def body(a0_ref, a1_ref, hs_ref, dinv_ref, b_ref, out_ref):
        out_ref[...] = ((a0_ref[...] + a1_ref[...] - hs_ref[...])
                        * dinv_ref[...] + b_ref[...])

    return pl.pallas_call(
        body,
        out_shape=jax.ShapeDtypeStruct((NP, D), jnp.float32),
    )(a0, a1, hs, dinv, b)


def kernel(x, edge_index, edge_attr, W1, b1, W2, b2, W3, b3):
    src = edge_index[0]
    dst = edge_index[1]
    b1r = b1.reshape(1, D)
    b2r = b2.reshape(1, D)
    b3r = b3.reshape(1, D)

    x_pad = jnp.pad(x, ((0, NP - N), (0, 0)))

    deg = _sc_deg(dst)
    d0 = deg[:NP].reshape(NP, 1)
    d1 = deg[NP:].reshape(NP, 1)

    hs1, dinv = _tc_first(x_pad, W1, d0, d1)
    acc = _sc_scatter(hs1, src, dst)
    hs2 = _tc_mid(acc[:NP], acc[NP:], hs1, dinv, b1r, W2)
    acc = _sc_scatter(hs2, src, dst)
    hs3 = _tc_mid(acc[:NP], acc[NP:], hs2, dinv, b2r, W3)
    acc = _sc_scatter(hs3, src, dst)
    out = _tc_final(acc[:NP], acc[NP:], hs3, dinv, b3r)
    return out[:N]


# speedup vs baseline: 26.8456x; 1.0500x over previous
import functools

import jax
import jax.numpy as jnp
from jax import lax
from jax.experimental import pallas as pl
from jax.experimental.pallas import tpu as pltpu
from jax.experimental.pallas import tpu_sc as plsc

N = 10000
NP = 10240
E = 320000
D = 128

NC = 2
NS = 16
NW = NC * NS
EPW = E // NW
CH = 80
NCHUNK = EPW // CH
RPT = NP // NS

DPT = 640
NDEG = NP

_sc_mesh = plsc.VectorSubcoreMesh(core_axis_name="c", subcore_axis_name="s")


@functools.partial(
    pl.kernel,
    mesh=_sc_mesh,
    out_type=jax.ShapeDtypeStruct((NC * NDEG,), jnp.float32),
    scratch_types=[
        pltpu.VMEM((4, CH), jnp.int32),
        pltpu.VMEM((DPT,), jnp.float32),
        pltpu.VMEM((CH,), jnp.float32),
        pltpu.VMEM_SHARED((NDEG,), jnp.float32),
        pltpu.SemaphoreType.DMA,
        pltpu.SemaphoreType.DMA,
    ],
)
def _sc_deg(dst_hbm, out_hbm, dstv, zbuf, ones, dacc, semi, sems):
    c = lax.axis_index("c")
    s = lax.axis_index("s")

    def fill(i, carry):
        zbuf[pl.ds(i * 16, 16)] = jnp.zeros((16,), jnp.float32)
        return carry
    lax.fori_loop(0, DPT // 16, fill, 0)

    def fill1(i, carry):
        ones[pl.ds(i * 16, 16)] = jnp.ones((16,), jnp.float32)
        return carry
    lax.fori_loop(0, CH // 16, fill1, 0)

    slab = pl.ds(s * DPT, DPT)
    pltpu.sync_copy(zbuf, dacc.at[slab])
    plsc.subcore_barrier()

    base = (c * NS + s) * EPW

    def _issue_idx(j, r):
        pltpu.async_copy(dst_hbm.at[pl.ds(base + j * CH, CH)],
                         dstv.at[r], semi)

    def _idx_wait():
        pltpu.make_async_copy(dst_hbm.at[pl.ds(base, CH)],
                              dstv.at[0], semi).wait()

    def _scatter_wait():
        pltpu.make_async_copy(ones, dacc.at[dstv.at[0]], sems).wait()

    _issue_idx(0, 0)
    _issue_idx(1, 1)

    def body(j, carry):
        r = lax.rem(j, 4)
        _idx_wait()
        pltpu.async_copy(ones, dacc.at[dstv.at[r]], sems, add=True)

        @pl.when(j >= 2)
        def _():
            _scatter_wait()

        @pl.when(j < NCHUNK - 2)
        def _():
            _issue_idx(j + 2, lax.rem(j + 2, 4))
        return carry
    lax.fori_loop(0, NCHUNK, body, 0)
    _scatter_wait()
    _scatter_wait()

    plsc.subcore_barrier()
    pltpu.sync_copy(dacc.at[slab], out_hbm.at[pl.ds(c * NDEG + s * DPT, DPT)])


@functools.partial(
    pl.kernel,
    mesh=_sc_mesh,
    out_type=jax.ShapeDtypeStruct((NC * NP, D), jnp.float32),
    scratch_types=[
        pltpu.VMEM((8, CH), jnp.int32),
        pltpu.VMEM((8, CH), jnp.int32),
        pltpu.VMEM((4, CH, D), jnp.float32),
        pltpu.VMEM_SHARED((NP, D), jnp.float32),
        pltpu.SemaphoreType.DMA,
        pltpu.SemaphoreType.DMA,
        pltpu.SemaphoreType.DMA,
    ],
)
def _sc_scatter(hs_hbm, src_hbm, dst_hbm, out_hbm, srcv, dstv, rows, acc,
                semg, semi, sems):
    c = lax.axis_index("c")
    s = lax.axis_index("s")
    base = (c * NS + s) * EPW

    slab = pl.ds(s * RPT, RPT)
    pltpu.sync_copy(hs_hbm.at[slab], acc.at[slab])
    plsc.subcore_barrier()

    def _issue_idx(j, r):
        pltpu.async_copy(src_hbm.at[pl.ds(base + j * CH, CH)],
                         srcv.at[r], semi)
        pltpu.async_copy(dst_hbm.at[pl.ds(base + j * CH, CH)],
                         dstv.at[r], semi)

    def _idx_wait():
        pltpu.make_async_copy(src_hbm.at[pl.ds(base, CH)],
                              srcv.at[0], semi).wait()
        pltpu.make_async_copy(dst_hbm.at[pl.ds(base, CH)],
                              dstv.at[0], semi).wait()

    def _gather_wait():
        pltpu.make_async_copy(hs_hbm.at[srcv.at[0]], rows.at[0], semg).wait()

    def _scatter_wait():
        pltpu.make_async_copy(rows.at[0], acc.at[dstv.at[0]], sems).wait()

    _issue_idx(0, 0)
    _issue_idx(1, 1)
    _issue_idx(2, 2)
    _issue_idx(3, 3)
    _idx_wait()
    pltpu.async_copy(hs_hbm.at[srcv.at[0]], rows.at[0], semg)
    _idx_wait()
    pltpu.async_copy(hs_hbm.at[srcv.at[1]], rows.at[1], semg)

    def body(j, carry):
        p = lax.rem(j, 4)
        r = lax.rem(j, 8)
        _gather_wait()
        pltpu.async_copy(rows.at[p], acc.at[dstv.at[r]], sems, add=True)

        @pl.when(j >= 2)
        def _():
            _scatter_wait()

        @pl.when(j < NCHUNK - 2)
        def _():
            _idx_wait()
            pltpu.async_copy(hs_hbm.at[srcv.at[lax.rem(j + 2, 8)]],
                             rows.at[lax.rem(j + 2, 4)], semg)

        @pl.when(j < NCHUNK - 4)
        def _():
            _issue_idx(j + 4, lax.rem(j + 4, 8))
        return carry
    lax.fori_loop(0, NCHUNK, body, 0)
    _scatter_wait()
    _scatter_wait()

    plsc.subcore_barrier()
    pltpu.sync_copy(acc.at[slab], out_hbm.at[pl.ds(c * NP + s * RPT, RPT)])


def _tc_mm1(x, W1):
    def body(x_ref, w_ref, h_ref):
        h_ref[...] = jnp.dot(x_ref[...], w_ref[...],
                             preferred_element_type=jnp.float32,
                             precision=lax.Precision.HIGHEST)

    return pl.pallas_call(
        body,
        out_shape=jax.ShapeDtypeStruct((NP, D), jnp.float32),
    )(x, W1)


def _tc_scale(h1, d0, d1):
    def body(h_ref, d0_ref, d1_ref, hs_ref, dinv_ref):
        deg = d0_ref[...] + d1_ref[...] + 1.0
        dinv = lax.rsqrt(deg)
        dinv_ref[...] = dinv
        hs_ref[...] = h_ref[...] * dinv

    return pl.pallas_call(
        body,
        out_shape=(jax.ShapeDtypeStruct((NP, D), jnp.float32),
                   jax.ShapeDtypeStruct((NP, 1), jnp.float32)),
    )(h1, d0, d1)


def _tc_mid(a0, a1, hs, dinv, b, W):
    def body(a0_ref, a1_ref, hs_ref, dinv_ref, b_ref, w_ref, out_ref):
        dinv = dinv_ref[...]
        z = (a0_ref[...] + a1_ref[...] - hs_ref[...]) * dinv + b_ref[...]
        z = jnp.maximum(z, 0.0)
        out_ref[...] = jnp.dot(z, w_ref[...],
                               preferred_element_type=jnp.float32,
                               precision=lax.Precision.HIGHEST) * dinv

    return pl.pallas_call(
        body,
        out_shape=jax.ShapeDtypeStruct((NP, D), jnp.float32),
    )(a0, a1, hs, dinv, b, W)


def _tc_final(a0, a1, hs, dinv, b):
    def body(a0_ref, a1_ref, hs_ref, dinv_ref, b_ref, out_ref):
        out_ref[...] = ((a0_ref[...] + a1_ref[...] - hs_ref[...])
                        * dinv_ref[...] + b_ref[...])

    return pl.pallas_call(
        body,
        out_shape=jax.ShapeDtypeStruct((NP, D), jnp.float32),
    )(a0, a1, hs, dinv, b)


def kernel(x, edge_index, edge_attr, W1, b1, W2, b2, W3, b3):
    src = edge_index[0]
    dst = edge_index[1]
    b1r = b1.reshape(1, D)
    b2r = b2.reshape(1, D)
    b3r = b3.reshape(1, D)

    x_pad = jnp.pad(x, ((0, NP - N), (0, 0)))

    deg = _sc_deg(dst)
    h1 = _tc_mm1(x_pad, W1)
    d0 = deg[:NP].reshape(NP, 1)
    d1 = deg[NP:].reshape(NP, 1)

    hs1, dinv = _tc_scale(h1, d0, d1)
    acc = _sc_scatter(hs1, src, dst)
    hs2 = _tc_mid(acc[:NP], acc[NP:], hs1, dinv, b1r, W2)
    acc = _sc_scatter(hs2, src, dst)
    hs3 = _tc_mid(acc[:NP], acc[NP:], hs2, dinv, b2r, W3)
    acc = _sc_scatter(hs3, src, dst)
    out = _tc_final(acc[:NP], acc[NP:], hs3, dinv, b3r)
    return out[:N]


# speedup vs baseline: 28.4060x; 1.0581x over previous
import functools

import jax
import jax.numpy as jnp
from jax import lax
from jax.experimental import pallas as pl
from jax.experimental.pallas import tpu as pltpu
from jax.experimental.pallas import tpu_sc as plsc

N = 10000
NP = 10240
E = 320000
D = 128

NC = 2
NS = 16
NW = NC * NS
EPW = E // NW
CH = 80
NCHUNK = EPW // CH
RPT = NP // NS

DPT = 640
NDEG = NP

_sc_mesh = plsc.VectorSubcoreMesh(core_axis_name="c", subcore_axis_name="s")


@functools.partial(
    pl.kernel,
    mesh=_sc_mesh,
    out_type=jax.ShapeDtypeStruct((NC * NDEG,), jnp.float32),
    scratch_types=[
        pltpu.VMEM((4, CH), jnp.int32),
        pltpu.VMEM((DPT,), jnp.float32),
        pltpu.VMEM((CH,), jnp.float32),
        pltpu.VMEM_SHARED((NDEG,), jnp.float32),
        pltpu.SemaphoreType.DMA,
        pltpu.SemaphoreType.DMA,
    ],
)
def _sc_deg(dst_hbm, out_hbm, dstv, zbuf, ones, dacc, semi, sems):
    c = lax.axis_index("c")
    s = lax.axis_index("s")

    def fill(i, carry):
        zbuf[pl.ds(i * 16, 16)] = jnp.zeros((16,), jnp.float32)
        return carry
    lax.fori_loop(0, DPT // 16, fill, 0)

    def fill1(i, carry):
        ones[pl.ds(i * 16, 16)] = jnp.ones((16,), jnp.float32)
        return carry
    lax.fori_loop(0, CH // 16, fill1, 0)

    slab = pl.ds(s * DPT, DPT)
    pltpu.sync_copy(zbuf, dacc.at[slab])
    plsc.subcore_barrier()

    base = (c * NS + s) * EPW

    def _issue_idx(j, r):
        pltpu.async_copy(dst_hbm.at[pl.ds(base + j * CH, CH)],
                         dstv.at[r], semi)

    def _idx_wait():
        pltpu.make_async_copy(dst_hbm.at[pl.ds(base, CH)],
                              dstv.at[0], semi).wait()

    def _scatter_wait():
        pltpu.make_async_copy(ones, dacc.at[dstv.at[0]], sems).wait()

    _issue_idx(0, 0)
    _issue_idx(1, 1)

    def body(j, carry):
        r = lax.rem(j, 4)
        _idx_wait()
        pltpu.async_copy(ones, dacc.at[dstv.at[r]], sems, add=True)

        @pl.when(j >= 2)
        def _():
            _scatter_wait()

        @pl.when(j < NCHUNK - 2)
        def _():
            _issue_idx(j + 2, lax.rem(j + 2, 4))
        return carry
    lax.fori_loop(0, NCHUNK, body, 0)
    _scatter_wait()
    _scatter_wait()

    plsc.subcore_barrier()
    pltpu.sync_copy(dacc.at[slab], out_hbm.at[pl.ds(c * NDEG + s * DPT, DPT)])


@functools.partial(
    pl.kernel,
    mesh=_sc_mesh,
    out_type=jax.ShapeDtypeStruct((NC * NP, D), jnp.float32),
    scratch_types=[
        pltpu.VMEM((8, CH), jnp.int32),
        pltpu.VMEM((8, CH), jnp.int32),
        pltpu.VMEM((4, CH, D), jnp.float32),
        pltpu.VMEM_SHARED((NP, D), jnp.float32),
        pltpu.SemaphoreType.DMA,
        pltpu.SemaphoreType.DMA,
        pltpu.SemaphoreType.DMA,
    ],
)
def _sc_scatter(hs_hbm, src_hbm, dst_hbm, out_hbm, srcv, dstv, rows, acc,
                semg, semi, sems):
    c = lax.axis_index("c")
    s = lax.axis_index("s")
    base = (c * NS + s) * EPW

    slab = pl.ds(s * RPT, RPT)
    pltpu.sync_copy(hs_hbm.at[slab], acc.at[slab])
    plsc.subcore_barrier()

    def _issue_idx(j, r):
        pltpu.async_copy(src_hbm.at[pl.ds(base + j * CH, CH)],
                         srcv.at[r], semi)
        pltpu.async_copy(dst_hbm.at[pl.ds(base + j * CH, CH)],
                         dstv.at[r], semi)

    def _idx_wait():
        pltpu.make_async_copy(src_hbm.at[pl.ds(base, CH)],
                              srcv.at[0], semi).wait()
        pltpu.make_async_copy(dst_hbm.at[pl.ds(base, CH)],
                              dstv.at[0], semi).wait()

    def _gather_wait():
        pltpu.make_async_copy(hs_hbm.at[srcv.at[0]], rows.at[0], semg).wait()

    def _scatter_wait():
        pltpu.make_async_copy(rows.at[0], acc.at[dstv.at[0]], sems).wait()

    _issue_idx(0, 0)
    _issue_idx(1, 1)
    _issue_idx(2, 2)
    _issue_idx(3, 3)
    _idx_wait()
    pltpu.async_copy(hs_hbm.at[srcv.at[0]], rows.at[0], semg)
    _idx_wait()
    pltpu.async_copy(hs_hbm.at[srcv.at[1]], rows.at[1], semg)

    def body(j, carry):
        p = lax.rem(j, 4)
        r = lax.rem(j, 8)
        _gather_wait()
        pltpu.async_copy(rows.at[p], acc.at[dstv.at[r]], sems, add=True)

        @pl.when(j >= 2)
        def _():
            _scatter_wait()

        @pl.when(j < NCHUNK - 2)
        def _():
            _idx_wait()
            pltpu.async_copy(hs_hbm.at[srcv.at[lax.rem(j + 2, 8)]],
                             rows.at[lax.rem(j + 2, 4)], semg)

        @pl.when(j < NCHUNK - 4)
        def _():
            _issue_idx(j + 4, lax.rem(j + 4, 8))
        return carry
    lax.fori_loop(0, NCHUNK, body, 0)
    _scatter_wait()
    _scatter_wait()

    plsc.subcore_barrier()
    pltpu.sync_copy(acc.at[slab], out_hbm.at[pl.ds(c * NP + s * RPT, RPT)])


def _tc_mm1(x, W1):
    def body(x_ref, w_ref, h_ref):
        h_ref[...] = jnp.dot(x_ref[...], w_ref[...],
                             preferred_element_type=jnp.float32,
                             precision=lax.Precision.HIGHEST)

    return pl.pallas_call(
        body,
        out_shape=jax.ShapeDtypeStruct((NP, D), jnp.float32),
    )(x, W1)


def _tc_scale(h1, d0, d1):
    def body(h_ref, d0_ref, d1_ref, hs_ref, dinv_ref):
        deg = d0_ref[...] + d1_ref[...] + 1.0
        dinv = lax.rsqrt(deg)
        dinv_ref[...] = dinv
        hs_ref[...] = h_ref[...] * dinv

    return pl.pallas_call(
        body,
        out_shape=(jax.ShapeDtypeStruct((NP, D), jnp.float32),
                   jax.ShapeDtypeStruct((NP, 1), jnp.float32)),
    )(h1, d0, d1)


def _tc_mid(acc, hs, dinv, b, W):
    def body(a_ref, hs_ref, dinv_ref, b_ref, w_ref, out_ref):
        dinv = dinv_ref[...]
        z = ((a_ref[:NP, :] + a_ref[NP:, :] - hs_ref[...]) * dinv
             + b_ref[...])
        z = jnp.maximum(z, 0.0)
        out_ref[...] = jnp.dot(z, w_ref[...],
                               preferred_element_type=jnp.float32,
                               precision=lax.Precision.HIGHEST) * dinv

    return pl.pallas_call(
        body,
        out_shape=jax.ShapeDtypeStruct((NP, D), jnp.float32),
    )(acc, hs, dinv, b, W)


def _tc_final(acc, hs, dinv, b):
    def body(a_ref, hs_ref, dinv_ref, b_ref, out_ref):
        out_ref[...] = ((a_ref[:N, :] + a_ref[NP:NP + N, :]
                         - hs_ref[:N, :]) * dinv_ref[:N, :] + b_ref[...])

    return pl.pallas_call(
        body,
        out_shape=jax.ShapeDtypeStruct((N, D), jnp.float32),
    )(acc, hs, dinv, b)


def kernel(x, edge_index, edge_attr, W1, b1, W2, b2, W3, b3):
    src = edge_index[0]
    dst = edge_index[1]
    b1r = b1.reshape(1, D)
    b2r = b2.reshape(1, D)
    b3r = b3.reshape(1, D)

    x_pad = jnp.pad(x, ((0, NP - N), (0, 0)))

    deg = _sc_deg(dst)
    h1 = _tc_mm1(x_pad, W1)
    d0 = deg[:NP].reshape(NP, 1)
    d1 = deg[NP:].reshape(NP, 1)

    hs1, dinv = _tc_scale(h1, d0, d1)
    acc = _sc_scatter(hs1, src, dst)
    hs2 = _tc_mid(acc, hs1, dinv, b1r, W2)
    acc = _sc_scatter(hs2, src, dst)
    hs3 = _tc_mid(acc, hs2, dinv, b2r, W3)
    acc = _sc_scatter(hs3, src, dst)
    return _tc_final(acc, hs3, dinv, b3r)
